# baseline (speedup 1.0000x reference)
import functools

import jax
import jax.numpy as jnp
import numpy as np
from jax import lax
from jax.experimental import pallas as pl
from jax.experimental.pallas import tpu as pltpu
from jax.experimental.pallas import tpu_sc as plsc

_LANES = 16
_NW = 32
_SLAB = 256
_TCHUNK = 2048
_RING = 2
_STG = 128
_CAP = 4096


def _make_gather(n_tok, vocab, dim, seq):
    mod = vocab - 1
    n_slabs = (vocab + _SLAB - 1) // _SLAB
    spw = n_slabs // _NW
    w31_slabs = n_slabs - (_NW - 1) * spw
    sbits = 23
    n_out = n_tok + _LANES
    mdim = 2 * dim
    mesh = plsc.VectorSubcoreMesh(core_axis_name="c", subcore_axis_name="s")

    @functools.partial(
        pl.kernel,
        mesh=mesh,
        out_type=jax.ShapeDtypeStruct((n_out, mdim), jnp.float32),
        scratch_types=[
            pltpu.VMEM((_TCHUNK,), jnp.int32),
            pltpu.VMEM((_TCHUNK,), jnp.int32),
            pltpu.VMEM((_CAP,), jnp.int32),
            pltpu.VMEM((_CAP,), jnp.int32),
            pltpu.VMEM((_RING, dim, _SLAB), jnp.float32),
            pltpu.VMEM((dim, 64), jnp.float32),
            pltpu.VMEM((2, _STG, mdim), jnp.float32),
            pltpu.VMEM((2, _STG), jnp.int32),
            pltpu.VMEM((128,), jnp.int32),
            pltpu.VMEM((128,), jnp.int32),
            pltpu.VMEM((128,), jnp.int32),
            pltpu.VMEM((_LANES,), jnp.int32),
            pltpu.VMEM((_LANES,), jnp.int32),
            pltpu.VMEM((_LANES,), jnp.int32),
            pltpu.SemaphoreType.DMA((_RING,)),
            pltpu.SemaphoreType.DMA((2,)),
        ],
        compiler_params=pltpu.CompilerParams(use_tc_tiling_on_sc=True,
                                             needs_layout_passes=False),
    )
    def gather_kernel(tok_hbm, tokp_hbm, tableT_hbm, tlast_hbm, h2_hbm,
                      tokc_v, tokp_v, comp_v, buck_v, ring_v, mini_v, stg_v, stgt_v,
                      counts_v, offs_v, cursor_v, tmps_v, tmpv_v, tmpm_v,
                      ssem, csem):
        wid = lax.axis_index("s") * 2 + lax.axis_index("c")
        start_slab = wid * spw
        is_last = wid == (_NW - 1)
        r_lo = start_slab * _SLAB
        r_hi = jnp.where(is_last, n_slabs * _SLAB, r_lo + spw * _SLAB)
        iota = lax.iota(jnp.int32, _LANES)
        zi = jnp.zeros((_LANES,), jnp.int32)
        zf = jnp.zeros((_LANES,), jnp.float32)
        ones = jnp.ones((_LANES,), jnp.int32)
        lane0 = iota == 0
        modv = jnp.full((_LANES,), mod, dtype=jnp.int32)
        dummy_t = n_tok + iota

        for b in range(128 // _LANES):
            counts_v[pl.ds(b * _LANES, _LANES)] = zi
        for sb in range(2):
            for rr in range(_STG):
                for cc in range(dim // _LANES):
                    stg_v[sb, rr, pl.ds(dim + cc * _LANES, _LANES)] = zf
            for cc in range(_STG // _LANES):
                stgt_v[sb, pl.ds(cc * _LANES, _LANES)] = dummy_t

        def chunk_body(ch, cnt):
            pltpu.sync_copy(tok_hbm.at[pl.ds(ch * _TCHUNK, _TCHUNK)], tokc_v)
            pltpu.sync_copy(tokp_hbm.at[pl.ds(ch * _TCHUNK, _TCHUNK)], tokp_v)

            def grp(i, cnt):
                cur = plsc.load_gather(tokc_v, [i * _LANES + iota])
                prev = plsc.load_gather(tokp_v, [i * _LANES + iota])
                h = (cur * 36313) ^ (prev * 27191)
                h = lax.rem(h, modv)
                pos = ch * _TCHUNK + i * _LANES + iota
                h = jnp.where((pos & (seq - 1)) == 0, mod, h)
                m = (h >= r_lo) & (h < r_hi)
                packed = ((h - r_lo) << 15) | pos
                plsc.store_compressed(comp_v.at[pl.ds(cnt, _LANES)], packed,
                                      mask=m)
                cnt = cnt + jnp.sum(m.astype(jnp.int32), dtype=jnp.int32)
                return jnp.minimum(cnt, _CAP - _LANES)

            return lax.fori_loop(jnp.int32(0), jnp.int32(_TCHUNK // _LANES),
                                 grp, cnt)

        n_local = lax.fori_loop(jnp.int32(0), jnp.int32(n_tok // _TCHUNK),
                                chunk_body, jnp.int32(0))

        def cb(g, _):
            lid = g * _LANES + iota
            m = lid < n_local
            v = plsc.load_gather(comp_v, [jnp.where(m, lid, 0)])
            s = (v >> sbits) & 127
            plsc.addupdate_scatter(counts_v, [s], ones, mask=m)
            return ()

        lax.fori_loop(jnp.int32(0), (n_local + _LANES - 1) >> 4, cb, ())

        carry = jnp.int32(0)
        for b in range(128 // _LANES):
            c = counts_v[pl.ds(b * _LANES, _LANES)]
            cs = plsc.cumsum(c)
            offs_v[pl.ds(b * _LANES, _LANES)] = cs - c + carry
            carry = carry + jnp.sum(c, dtype=jnp.int32)
        for b in range(128 // _LANES):
            cursor_v[pl.ds(b * _LANES, _LANES)] = offs_v[pl.ds(b * _LANES,
                                                               _LANES)]

        def pgrp(g, _):
            lid = g * _LANES + iota
            m = lid < n_local
            v = plsc.load_gather(comp_v, [jnp.where(m, lid, 0)])
            tmps_v[pl.ds(0, _LANES)] = (v >> sbits) & 127
            tmpv_v[pl.ds(0, _LANES)] = v
            tmpm_v[pl.ds(0, _LANES)] = m.astype(jnp.int32)
            for l in range(_LANES):
                li = jnp.full((_LANES,), l, jnp.int32)
                sl_ = plsc.load_gather(tmps_v, [li])
                vl = plsc.load_gather(tmpv_v, [li])
                ml = plsc.load_gather(tmpm_v, [li])
                p = plsc.load_gather(cursor_v, [sl_])
                wm = lane0 & (ml > 0) & (p < _CAP)
                plsc.store_scatter(buck_v, [p], vl, mask=wm)
                plsc.store_scatter(cursor_v, [sl_], p + 1, mask=wm)
            return ()

        lax.fori_loop(jnp.int32(0), (n_local + _LANES - 1) >> 4, pgrp, ())

        pltpu.async_copy(stg_v.at[jnp.int32(0)],
                         h2_hbm.at[stgt_v.at[jnp.int32(0)]],
                         csem.at[jnp.int32(0)])
        pltpu.async_copy(stg_v.at[jnp.int32(1)],
                         h2_hbm.at[stgt_v.at[jnp.int32(1)]],
                         csem.at[jnp.int32(1)])
        pltpu.make_async_copy(h2_hbm.at[pl.ds(0, _STG)],
                              stg_v.at[jnp.int32(0)],
                              csem.at[jnp.int32(0)]).wait()

        def reset_stgt(par):
            for cc in range(_STG // _LANES):
                plsc.store_scatter(
                    stgt_v, [jnp.full((_LANES,), par, jnp.int32),
                             cc * _LANES + iota], dummy_t)

        def extract_bucket(sl, from_mini, carry):
            slv = jnp.full((_LANES,), sl, jnp.int32)
            off = plsc.load_gather(offs_v, [slv])
            cntb = plsc.load_gather(counts_v, [slv])
            cnt_sc = jnp.minimum(jnp.max(cntb), jnp.int32(_CAP))
            bb = jnp.full((_LANES,), sl & (_RING - 1), jnp.int32)

            def egrp(g, carry):
                fill, nf = carry
                par = nf & 1
                lid = off + g * _LANES + iota
                m = (g * _LANES + iota) < cntb
                v = plsc.load_gather(buck_v, [jnp.where(m, lid & (_CAP - 1),
                                                        0)])
                v = jnp.where(m, v, 0)
                col = jnp.where(m, (v >> 15) - sl * _SLAB, 0)
                tsc = jnp.where(m, v & 32767, dummy_t)
                parv = jnp.full((_LANES,), par, jnp.int32)
                rows = fill + iota
                plsc.store_scatter(stgt_v, [parv, rows], tsc)
                for f in range(dim):
                    fv = jnp.full((_LANES,), f, jnp.int32)
                    if from_mini:
                        vals = plsc.load_gather(mini_v, [fv, col])
                    else:
                        vals = plsc.load_gather(ring_v, [bb, fv, col])
                    plsc.store_scatter(stg_v, [parv, rows, fv], vals)
                fill = fill + _LANES
                do = fill >= _STG

                def do_flush():
                    pltpu.async_copy(stg_v.at[par],
                                     h2_hbm.at[stgt_v.at[par]], csem.at[par])
                    op = 1 - par
                    pltpu.make_async_copy(h2_hbm.at[pl.ds(0, _STG)],
                                          stg_v.at[op], csem.at[op]).wait()
                    reset_stgt(op)
                    return jnp.int32(0)

                lax.cond(do, do_flush, lambda: jnp.int32(0))
                nf2 = nf + do.astype(jnp.int32)
                fill = jnp.where(do, 0, fill)
                return (fill, nf2)

            return lax.fori_loop(jnp.int32(0), (cnt_sc + _LANES - 1) >> 4,
                                 egrp, carry)

        n_full = jnp.where(is_last, jnp.int32(w31_slabs - 1), jnp.int32(spw))

        def fetch(sl):
            j = sl & (_RING - 1)
            pltpu.async_copy(
                tableT_hbm.at[:, pl.ds((start_slab + sl) * _SLAB, _SLAB)],
                ring_v.at[j], ssem.at[j])

        def prime(k, _):
            @pl.when(k < n_full)
            def _():
                fetch(k)
            return ()

        lax.fori_loop(jnp.int32(0), jnp.int32(_RING), prime, ())

        def slab_body(sl, carry):
            j = sl & (_RING - 1)
            pltpu.make_async_copy(tableT_hbm.at[:, pl.ds(0, _SLAB)],
                                  ring_v.at[j], ssem.at[j]).wait()
            carry = extract_bucket(sl, False, carry)

            @pl.when(sl + _RING < n_full)
            def _():
                fetch(sl + _RING)

            return carry

        fill, nf = lax.fori_loop(jnp.int32(0), n_full, slab_body,
                                 (jnp.int32(0), jnp.int32(0)))

        def mini_extract(carry):
            pltpu.sync_copy(tlast_hbm, mini_v)
            return extract_bucket(jnp.int32(w31_slabs - 1), True, carry)

        fill, nf = lax.cond(is_last, lambda: mini_extract((fill, nf)),
                            lambda: (fill, nf))

        par = nf & 1

        @pl.when(fill > 0)
        def _():
            pltpu.async_copy(stg_v.at[par], h2_hbm.at[stgt_v.at[par]],
                             csem.at[par])

        w0 = (par == 1) | ((fill > 0) & (par == 0))
        w1 = par == 0

        @pl.when(w0)
        def _():
            pltpu.make_async_copy(h2_hbm.at[pl.ds(0, _STG)],
                                  stg_v.at[jnp.int32(0)],
                                  csem.at[jnp.int32(0)]).wait()

        @pl.when(w1)
        def _():
            pltpu.make_async_copy(h2_hbm.at[pl.ds(0, _STG)],
                                  stg_v.at[jnp.int32(1)],
                                  csem.at[jnp.int32(1)]).wait()

    return gather_kernel


_ZERO = np.int32(0)


def _mm_body(h_ref, w_ref, scale_ref, o_ref):
    o_ref[...] = lax.dot_general(
        h_ref[...][:, :w_ref.shape[1]], w_ref[...],
        dimension_numbers=(((1,), (1,)), ((), ())),
        preferred_element_type=jnp.float32) * scale_ref[0]


def _make_matmul(n_tok, mdim, model_dim, block_rows=1024):
    grid = n_tok // block_rows
    return pl.pallas_call(
        _mm_body,
        grid=(grid,),
        in_specs=[
            pl.BlockSpec((block_rows, mdim), lambda i: (i, _ZERO)),
            pl.BlockSpec((model_dim, mdim // 2), lambda i: (_ZERO, _ZERO)),
            pl.BlockSpec((1,), lambda i: (_ZERO,), memory_space=pltpu.SMEM),
        ],
        out_specs=pl.BlockSpec((block_rows, model_dim), lambda i: (i, _ZERO)),
        out_shape=jax.ShapeDtypeStruct((n_tok, model_dim), jnp.float32),
    )


def kernel(token_ids, table, W_proj, scale):
    batch, seq = token_ids.shape
    vocab, dim = table.shape
    model_dim = W_proj.shape[0]
    n_tok = batch * seq

    tok = token_ids.astype(jnp.int32).reshape(-1)
    tok_prev = jnp.concatenate([jnp.zeros((1,), jnp.int32), tok[:-1]])
    tableT = table.T
    n_slabs = (vocab + _SLAB - 1) // _SLAB
    tlast = tableT[:, (n_slabs - 1) * _SLAB:]

    h2 = _make_gather(n_tok, vocab, dim, seq)(tok, tok_prev, tableT, tlast)
    scale1 = jnp.reshape(scale, (1,)).astype(jnp.float32)
    out = _make_matmul(n_tok, 2 * dim, model_dim)(h2[:n_tok], W_proj, scale1)
    return out.reshape(batch, seq, model_dim)


# speedup vs baseline: 1.7864x; 1.7864x over previous
import functools

import jax
import jax.numpy as jnp
import numpy as np
from jax import lax
from jax.experimental import pallas as pl
from jax.experimental.pallas import tpu as pltpu
from jax.experimental.pallas import tpu_sc as plsc

_LANES = 16
_NW = 32
_SLAB = 256
_TCHUNK = 2048
_RING = 2
_STG = 128
_CAP = 4096


def _make_gather(n_tok, vocab, dim, seq):
    mod = vocab - 1
    n_slabs = (vocab + _SLAB - 1) // _SLAB
    spw = n_slabs // _NW
    w31_slabs = n_slabs - (_NW - 1) * spw
    sbits = 23
    n_out = n_tok + _LANES
    mdim = 2 * dim
    mesh = plsc.VectorSubcoreMesh(core_axis_name="c", subcore_axis_name="s")

    @functools.partial(
        pl.kernel,
        mesh=mesh,
        out_type=jax.ShapeDtypeStruct((n_out, mdim), jnp.float32),
        scratch_types=[
            pltpu.VMEM((_TCHUNK,), jnp.int32),
            pltpu.VMEM((_TCHUNK,), jnp.int32),
            pltpu.VMEM((_CAP,), jnp.int32),
            pltpu.VMEM((_CAP,), jnp.int32),
            pltpu.VMEM((_RING, dim, _SLAB), jnp.float32),
            pltpu.VMEM((dim, 64), jnp.float32),
            pltpu.VMEM((2, _STG, mdim), jnp.float32),
            pltpu.VMEM((2, _STG), jnp.int32),
            pltpu.VMEM((128,), jnp.int32),
            pltpu.VMEM((128,), jnp.int32),
            pltpu.VMEM((128,), jnp.int32),
            pltpu.VMEM((_LANES,), jnp.int32),
            pltpu.VMEM((_LANES,), jnp.int32),
            pltpu.VMEM((_LANES,), jnp.int32),
            pltpu.SemaphoreType.DMA((_RING,)),
            pltpu.SemaphoreType.DMA((2,)),
        ],
        compiler_params=pltpu.CompilerParams(use_tc_tiling_on_sc=True,
                                             needs_layout_passes=False),
    )
    def gather_kernel(tok_hbm, tokp_hbm, tableT_hbm, tlast_hbm, h2_hbm,
                      tokc_v, tokp_v, comp_v, buck_v, ring_v, mini_v, stg_v, stgt_v,
                      counts_v, offs_v, cursor_v, tmps_v, tmpv_v, tmpm_v,
                      ssem, csem):
        wid = lax.axis_index("s") * 2 + lax.axis_index("c")
        start_slab = wid * spw
        is_last = wid == (_NW - 1)
        r_lo = start_slab * _SLAB
        r_hi = jnp.where(is_last, n_slabs * _SLAB, r_lo + spw * _SLAB)
        iota = lax.iota(jnp.int32, _LANES)
        zi = jnp.zeros((_LANES,), jnp.int32)
        zf = jnp.zeros((_LANES,), jnp.float32)
        ones = jnp.ones((_LANES,), jnp.int32)
        lane0 = iota == 0
        modv = jnp.full((_LANES,), mod, dtype=jnp.int32)
        dummy_t = n_tok + iota

        for b in range(128 // _LANES):
            counts_v[pl.ds(b * _LANES, _LANES)] = zi
        for sb in range(2):
            for rr in range(_STG):
                for cc in range(dim // _LANES):
                    stg_v[sb, rr, pl.ds(dim + cc * _LANES, _LANES)] = zf
            for cc in range(_STG // _LANES):
                stgt_v[sb, pl.ds(cc * _LANES, _LANES)] = dummy_t

        def chunk_body(ch, cnt):
            pltpu.sync_copy(tok_hbm.at[pl.ds(ch * _TCHUNK, _TCHUNK)], tokc_v)
            pltpu.sync_copy(tokp_hbm.at[pl.ds(ch * _TCHUNK, _TCHUNK)], tokp_v)

            def grp(i, cnt):
                cur = plsc.load_gather(tokc_v, [i * _LANES + iota])
                prev = plsc.load_gather(tokp_v, [i * _LANES + iota])
                h = (cur * 36313) ^ (prev * 27191)
                h = lax.rem(h, modv)
                pos = ch * _TCHUNK + i * _LANES + iota
                h = jnp.where((pos & (seq - 1)) == 0, mod, h)
                m = (h >= r_lo) & (h < r_hi)
                packed = ((h - r_lo) << 15) | pos
                plsc.store_compressed(comp_v.at[pl.ds(cnt, _LANES)], packed,
                                      mask=m)
                cnt = cnt + jnp.sum(m.astype(jnp.int32), dtype=jnp.int32)
                return jnp.minimum(cnt, _CAP - _LANES)

            return lax.fori_loop(jnp.int32(0), jnp.int32(_TCHUNK // _LANES),
                                 grp, cnt)

        n_local = lax.fori_loop(jnp.int32(0), jnp.int32(n_tok // _TCHUNK),
                                chunk_body, jnp.int32(0))

        def cb(g, _):
            lid = g * _LANES + iota
            m = lid < n_local
            v = plsc.load_gather(comp_v, [jnp.where(m, lid, 0)])
            s = (v >> sbits) & 127
            plsc.addupdate_scatter(counts_v, [s], ones, mask=m)
            return ()

        lax.fori_loop(jnp.int32(0), (n_local + _LANES - 1) >> 4, cb, ())

        carry = jnp.int32(0)
        for b in range(128 // _LANES):
            c = counts_v[pl.ds(b * _LANES, _LANES)]
            cs = plsc.cumsum(c)
            offs_v[pl.ds(b * _LANES, _LANES)] = cs - c + carry
            carry = carry + jnp.sum(c, dtype=jnp.int32)
        for b in range(128 // _LANES):
            cursor_v[pl.ds(b * _LANES, _LANES)] = offs_v[pl.ds(b * _LANES,
                                                               _LANES)]

        def pgrp(g, _):
            lid = g * _LANES + iota
            m = lid < n_local
            v = plsc.load_gather(comp_v, [jnp.where(m, lid, 0)])
            tmps_v[pl.ds(0, _LANES)] = (v >> sbits) & 127
            tmpv_v[pl.ds(0, _LANES)] = v
            tmpm_v[pl.ds(0, _LANES)] = m.astype(jnp.int32)
            for l in range(_LANES):
                li = jnp.full((_LANES,), l, jnp.int32)
                sl_ = plsc.load_gather(tmps_v, [li])
                vl = plsc.load_gather(tmpv_v, [li])
                ml = plsc.load_gather(tmpm_v, [li])
                p = plsc.load_gather(cursor_v, [sl_])
                wm = lane0 & (ml > 0) & (p < _CAP)
                plsc.store_scatter(buck_v, [p], vl, mask=wm)
                plsc.store_scatter(cursor_v, [sl_], p + 1, mask=wm)
            return ()

        lax.fori_loop(jnp.int32(0), (n_local + _LANES - 1) >> 4, pgrp, ())

        pltpu.async_copy(stg_v.at[jnp.int32(0)],
                         h2_hbm.at[stgt_v.at[jnp.int32(0)]],
                         csem.at[jnp.int32(0)])
        pltpu.async_copy(stg_v.at[jnp.int32(1)],
                         h2_hbm.at[stgt_v.at[jnp.int32(1)]],
                         csem.at[jnp.int32(1)])
        pltpu.make_async_copy(h2_hbm.at[pl.ds(0, _STG)],
                              stg_v.at[jnp.int32(0)],
                              csem.at[jnp.int32(0)]).wait()

        def reset_stgt(par):
            for cc in range(_STG // _LANES):
                plsc.store_scatter(
                    stgt_v, [jnp.full((_LANES,), par, jnp.int32),
                             cc * _LANES + iota], dummy_t)

        def extract_bucket(sl, from_mini, carry):
            slv = jnp.full((_LANES,), sl, jnp.int32)
            off = plsc.load_gather(offs_v, [slv])
            cntb = plsc.load_gather(counts_v, [slv])
            cnt_sc = jnp.minimum(jnp.max(cntb), jnp.int32(_CAP))
            bb = jnp.full((_LANES,), sl & (_RING - 1), jnp.int32)

            def ehit(k, carry):
                fill, nf = carry
                par = nf & 1
                parv = jnp.full((_LANES,), par, jnp.int32)
                v = plsc.load_gather(buck_v, [(off + k) & (_CAP - 1)])
                col = (v >> 15) - sl * _SLAB
                tval = v & 32767
                rowv = jnp.full((_LANES,), fill, jnp.int32)
                for cc in range(dim // _LANES):
                    f16 = cc * _LANES + iota
                    if from_mini:
                        vals = plsc.load_gather(mini_v, [f16, col])
                    else:
                        vals = plsc.load_gather(ring_v, [bb, f16, col])
                    plsc.store_scatter(stg_v, [parv, rowv, f16], vals)
                plsc.store_scatter(stgt_v, [parv, rowv], tval, mask=lane0)
                fill = fill + 1
                do = fill >= _STG

                def do_flush():
                    pltpu.async_copy(stg_v.at[par],
                                     h2_hbm.at[stgt_v.at[par]], csem.at[par])
                    op = 1 - par
                    pltpu.make_async_copy(h2_hbm.at[pl.ds(0, _STG)],
                                          stg_v.at[op], csem.at[op]).wait()
                    reset_stgt(op)
                    return jnp.int32(0)

                lax.cond(do, do_flush, lambda: jnp.int32(0))
                nf2 = nf + do.astype(jnp.int32)
                fill = jnp.where(do, 0, fill)
                return (fill, nf2)

            return lax.fori_loop(jnp.int32(0), cnt_sc, ehit, carry)

        n_full = jnp.where(is_last, jnp.int32(w31_slabs - 1), jnp.int32(spw))

        def fetch(sl):
            j = sl & (_RING - 1)
            pltpu.async_copy(
                tableT_hbm.at[:, pl.ds((start_slab + sl) * _SLAB, _SLAB)],
                ring_v.at[j], ssem.at[j])

        def prime(k, _):
            @pl.when(k < n_full)
            def _():
                fetch(k)
            return ()

        lax.fori_loop(jnp.int32(0), jnp.int32(_RING), prime, ())

        def slab_body(sl, carry):
            j = sl & (_RING - 1)
            pltpu.make_async_copy(tableT_hbm.at[:, pl.ds(0, _SLAB)],
                                  ring_v.at[j], ssem.at[j]).wait()
            carry = extract_bucket(sl, False, carry)

            @pl.when(sl + _RING < n_full)
            def _():
                fetch(sl + _RING)

            return carry

        fill, nf = lax.fori_loop(jnp.int32(0), n_full, slab_body,
                                 (jnp.int32(0), jnp.int32(0)))

        def mini_extract(carry):
            pltpu.sync_copy(tlast_hbm, mini_v)
            return extract_bucket(jnp.int32(w31_slabs - 1), True, carry)

        fill, nf = lax.cond(is_last, lambda: mini_extract((fill, nf)),
                            lambda: (fill, nf))

        par = nf & 1

        @pl.when(fill > 0)
        def _():
            pltpu.async_copy(stg_v.at[par], h2_hbm.at[stgt_v.at[par]],
                             csem.at[par])

        w0 = (par == 1) | ((fill > 0) & (par == 0))
        w1 = par == 0

        @pl.when(w0)
        def _():
            pltpu.make_async_copy(h2_hbm.at[pl.ds(0, _STG)],
                                  stg_v.at[jnp.int32(0)],
                                  csem.at[jnp.int32(0)]).wait()

        @pl.when(w1)
        def _():
            pltpu.make_async_copy(h2_hbm.at[pl.ds(0, _STG)],
                                  stg_v.at[jnp.int32(1)],
                                  csem.at[jnp.int32(1)]).wait()

    return gather_kernel


_ZERO = np.int32(0)


def _mm_body(h_ref, w_ref, scale_ref, o_ref):
    o_ref[...] = lax.dot_general(
        h_ref[...][:, :w_ref.shape[1]], w_ref[...],
        dimension_numbers=(((1,), (1,)), ((), ())),
        preferred_element_type=jnp.float32) * scale_ref[0]


def _make_matmul(n_tok, mdim, model_dim, block_rows=1024):
    grid = n_tok // block_rows
    return pl.pallas_call(
        _mm_body,
        grid=(grid,),
        in_specs=[
            pl.BlockSpec((block_rows, mdim), lambda i: (i, _ZERO)),
            pl.BlockSpec((model_dim, mdim // 2), lambda i: (_ZERO, _ZERO)),
            pl.BlockSpec((1,), lambda i: (_ZERO,), memory_space=pltpu.SMEM),
        ],
        out_specs=pl.BlockSpec((block_rows, model_dim), lambda i: (i, _ZERO)),
        out_shape=jax.ShapeDtypeStruct((n_tok, model_dim), jnp.float32),
    )


def kernel(token_ids, table, W_proj, scale):
    batch, seq = token_ids.shape
    vocab, dim = table.shape
    model_dim = W_proj.shape[0]
    n_tok = batch * seq

    tok = token_ids.astype(jnp.int32).reshape(-1)
    tok_prev = jnp.concatenate([jnp.zeros((1,), jnp.int32), tok[:-1]])
    tableT = table.T
    n_slabs = (vocab + _SLAB - 1) // _SLAB
    tlast = tableT[:, (n_slabs - 1) * _SLAB:]

    h2 = _make_gather(n_tok, vocab, dim, seq)(tok, tok_prev, tableT, tlast)
    scale1 = jnp.reshape(scale, (1,)).astype(jnp.float32)
    out = _make_matmul(n_tok, 2 * dim, model_dim)(h2[:n_tok], W_proj, scale1)
    return out.reshape(batch, seq, model_dim)


# speedup vs baseline: 1.9560x; 1.0950x over previous
import functools

import jax
import jax.numpy as jnp
import numpy as np
from jax import lax
from jax.experimental import pallas as pl
from jax.experimental.pallas import tpu as pltpu
from jax.experimental.pallas import tpu_sc as plsc

_LANES = 16
_NW = 32
_SLAB = 512
_TCHUNK = 1024
_RING = 2
_STG = 64
_CAP = 2048


def _make_gather(n_tok, vocab, dim, seq):
    mod = vocab - 1
    n_slabs = (vocab + _SLAB - 1) // _SLAB
    spw = n_slabs // _NW
    w31_slabs = n_slabs - (_NW - 1) * spw
    sbits = 24
    n_out = n_tok + _LANES
    mdim = 2 * dim
    mesh = plsc.VectorSubcoreMesh(core_axis_name="c", subcore_axis_name="s")

    @functools.partial(
        pl.kernel,
        mesh=mesh,
        out_type=jax.ShapeDtypeStruct((n_out, mdim), jnp.float32),
        scratch_types=[
            pltpu.VMEM((_TCHUNK,), jnp.int32),
            pltpu.VMEM((_TCHUNK,), jnp.int32),
            pltpu.VMEM((_CAP,), jnp.int32),
            pltpu.VMEM((_CAP,), jnp.int32),
            pltpu.VMEM((_RING, dim, _SLAB), jnp.float32),
            pltpu.VMEM((dim, 64), jnp.float32),
            pltpu.VMEM((2, _STG, mdim), jnp.float32),
            pltpu.VMEM((2, _STG), jnp.int32),
            pltpu.VMEM((128,), jnp.int32),
            pltpu.VMEM((128,), jnp.int32),
            pltpu.VMEM((128,), jnp.int32),
            pltpu.VMEM((_LANES,), jnp.int32),
            pltpu.VMEM((_LANES,), jnp.int32),
            pltpu.VMEM((_LANES,), jnp.int32),
            pltpu.SemaphoreType.DMA((_RING,)),
            pltpu.SemaphoreType.DMA((2,)),
        ],
        compiler_params=pltpu.CompilerParams(use_tc_tiling_on_sc=True,
                                             needs_layout_passes=False),
    )
    def gather_kernel(tok_hbm, tokp_hbm, tableT_hbm, tlast_hbm, h2_hbm,
                      tokc_v, tokp_v, comp_v, buck_v, ring_v, mini_v, stg_v, stgt_v,
                      counts_v, offs_v, cursor_v, tmps_v, tmpv_v, tmpm_v,
                      ssem, csem):
        wid = lax.axis_index("s") * 2 + lax.axis_index("c")
        start_slab = wid * spw
        is_last = wid == (_NW - 1)
        r_lo = start_slab * _SLAB
        r_hi = jnp.where(is_last, n_slabs * _SLAB, r_lo + spw * _SLAB)
        iota = lax.iota(jnp.int32, _LANES)
        zi = jnp.zeros((_LANES,), jnp.int32)
        zf = jnp.zeros((_LANES,), jnp.float32)
        ones = jnp.ones((_LANES,), jnp.int32)
        lane0 = iota == 0
        modv = jnp.full((_LANES,), mod, dtype=jnp.int32)
        dummy_t = n_tok + iota

        for b in range(128 // _LANES):
            counts_v[pl.ds(b * _LANES, _LANES)] = zi
        for sb in range(2):
            for rr in range(_STG):
                for cc in range(dim // _LANES):
                    stg_v[sb, rr, pl.ds(dim + cc * _LANES, _LANES)] = zf
            for cc in range(_STG // _LANES):
                stgt_v[sb, pl.ds(cc * _LANES, _LANES)] = dummy_t

        def chunk_body(ch, cnt):
            pltpu.sync_copy(tok_hbm.at[pl.ds(ch * _TCHUNK, _TCHUNK)], tokc_v)
            pltpu.sync_copy(tokp_hbm.at[pl.ds(ch * _TCHUNK, _TCHUNK)], tokp_v)

            def grp(i, cnt):
                cur = plsc.load_gather(tokc_v, [i * _LANES + iota])
                prev = plsc.load_gather(tokp_v, [i * _LANES + iota])
                h = (cur * 36313) ^ (prev * 27191)
                h = lax.rem(h, modv)
                pos = ch * _TCHUNK + i * _LANES + iota
                h = jnp.where((pos & (seq - 1)) == 0, mod, h)
                m = (h >= r_lo) & (h < r_hi)
                packed = ((h - r_lo) << 15) | pos
                plsc.store_compressed(comp_v.at[pl.ds(cnt, _LANES)], packed,
                                      mask=m)
                cnt = cnt + jnp.sum(m.astype(jnp.int32), dtype=jnp.int32)
                return jnp.minimum(cnt, _CAP - _LANES)

            return lax.fori_loop(jnp.int32(0), jnp.int32(_TCHUNK // _LANES),
                                 grp, cnt)

        n_local = lax.fori_loop(jnp.int32(0), jnp.int32(n_tok // _TCHUNK),
                                chunk_body, jnp.int32(0))

        def cb(g, _):
            lid = g * _LANES + iota
            m = lid < n_local
            v = plsc.load_gather(comp_v, [jnp.where(m, lid, 0)])
            s = (v >> sbits) & 127
            plsc.addupdate_scatter(counts_v, [s], ones, mask=m)
            return ()

        lax.fori_loop(jnp.int32(0), (n_local + _LANES - 1) >> 4, cb, ())

        carry = jnp.int32(0)
        for b in range(128 // _LANES):
            c = counts_v[pl.ds(b * _LANES, _LANES)]
            cs = plsc.cumsum(c)
            offs_v[pl.ds(b * _LANES, _LANES)] = cs - c + carry
            carry = carry + jnp.sum(c, dtype=jnp.int32)
        for b in range(128 // _LANES):
            cursor_v[pl.ds(b * _LANES, _LANES)] = offs_v[pl.ds(b * _LANES,
                                                               _LANES)]

        def pgrp(g, _):
            lid = g * _LANES + iota
            m = lid < n_local
            v = plsc.load_gather(comp_v, [jnp.where(m, lid, 0)])
            tmps_v[pl.ds(0, _LANES)] = (v >> sbits) & 127
            tmpv_v[pl.ds(0, _LANES)] = v
            tmpm_v[pl.ds(0, _LANES)] = m.astype(jnp.int32)
            for l in range(_LANES):
                li = jnp.full((_LANES,), l, jnp.int32)
                sl_ = plsc.load_gather(tmps_v, [li])
                vl = plsc.load_gather(tmpv_v, [li])
                ml = plsc.load_gather(tmpm_v, [li])
                p = plsc.load_gather(cursor_v, [sl_])
                wm = lane0 & (ml > 0) & (p < _CAP)
                plsc.store_scatter(buck_v, [p], vl, mask=wm)
                plsc.store_scatter(cursor_v, [sl_], p + 1, mask=wm)
            return ()

        lax.fori_loop(jnp.int32(0), (n_local + _LANES - 1) >> 4, pgrp, ())

        pltpu.async_copy(stg_v.at[jnp.int32(0)],
                         h2_hbm.at[stgt_v.at[jnp.int32(0)]],
                         csem.at[jnp.int32(0)])
        pltpu.async_copy(stg_v.at[jnp.int32(1)],
                         h2_hbm.at[stgt_v.at[jnp.int32(1)]],
                         csem.at[jnp.int32(1)])
        pltpu.make_async_copy(h2_hbm.at[pl.ds(0, _STG)],
                              stg_v.at[jnp.int32(0)],
                              csem.at[jnp.int32(0)]).wait()

        def reset_stgt(par):
            for cc in range(_STG // _LANES):
                plsc.store_scatter(
                    stgt_v, [jnp.full((_LANES,), par, jnp.int32),
                             cc * _LANES + iota], dummy_t)

        def extract_bucket(sl, from_mini, carry):
            slv = jnp.full((_LANES,), sl, jnp.int32)
            off = plsc.load_gather(offs_v, [slv])
            cntb = plsc.load_gather(counts_v, [slv])
            cnt_sc = jnp.minimum(jnp.max(cntb), jnp.int32(_CAP))
            bb = jnp.full((_LANES,), sl & (_RING - 1), jnp.int32)

            def ehit(k, carry):
                fill, nf = carry
                par = nf & 1
                parv = jnp.full((_LANES,), par, jnp.int32)
                v = plsc.load_gather(buck_v, [(off + k) & (_CAP - 1)])
                col = (v >> 15) - sl * _SLAB
                tval = v & 32767
                rowv = jnp.full((_LANES,), fill, jnp.int32)
                for cc in range(dim // _LANES):
                    f16 = cc * _LANES + iota
                    if from_mini:
                        vals = plsc.load_gather(mini_v, [f16, col])
                    else:
                        vals = plsc.load_gather(ring_v, [bb, f16, col])
                    plsc.store_scatter(stg_v, [parv, rowv, f16], vals)
                plsc.store_scatter(stgt_v, [parv, rowv], tval, mask=lane0)
                fill = fill + 1
                do = fill >= _STG

                def do_flush():
                    pltpu.async_copy(stg_v.at[par],
                                     h2_hbm.at[stgt_v.at[par]], csem.at[par])
                    op = 1 - par
                    pltpu.make_async_copy(h2_hbm.at[pl.ds(0, _STG)],
                                          stg_v.at[op], csem.at[op]).wait()
                    reset_stgt(op)
                    return jnp.int32(0)

                lax.cond(do, do_flush, lambda: jnp.int32(0))
                nf2 = nf + do.astype(jnp.int32)
                fill = jnp.where(do, 0, fill)
                return (fill, nf2)

            return lax.fori_loop(jnp.int32(0), cnt_sc, ehit, carry)

        n_full = jnp.where(is_last, jnp.int32(w31_slabs - 1), jnp.int32(spw))

        def fetch(sl):
            j = sl & (_RING - 1)
            pltpu.async_copy(
                tableT_hbm.at[:, pl.ds((start_slab + sl) * _SLAB, _SLAB)],
                ring_v.at[j], ssem.at[j])

        def prime(k, _):
            @pl.when(k < n_full)
            def _():
                fetch(k)
            return ()

        lax.fori_loop(jnp.int32(0), jnp.int32(_RING), prime, ())

        def slab_body(sl, carry):
            j = sl & (_RING - 1)
            pltpu.make_async_copy(tableT_hbm.at[:, pl.ds(0, _SLAB)],
                                  ring_v.at[j], ssem.at[j]).wait()
            carry = extract_bucket(sl, False, carry)

            @pl.when(sl + _RING < n_full)
            def _():
                fetch(sl + _RING)

            return carry

        fill, nf = lax.fori_loop(jnp.int32(0), n_full, slab_body,
                                 (jnp.int32(0), jnp.int32(0)))

        def mini_extract(carry):
            pltpu.sync_copy(tlast_hbm, mini_v)
            return extract_bucket(jnp.int32(w31_slabs - 1), True, carry)

        fill, nf = lax.cond(is_last, lambda: mini_extract((fill, nf)),
                            lambda: (fill, nf))

        par = nf & 1

        @pl.when(fill > 0)
        def _():
            pltpu.async_copy(stg_v.at[par], h2_hbm.at[stgt_v.at[par]],
                             csem.at[par])

        w0 = (par == 1) | ((fill > 0) & (par == 0))
        w1 = par == 0

        @pl.when(w0)
        def _():
            pltpu.make_async_copy(h2_hbm.at[pl.ds(0, _STG)],
                                  stg_v.at[jnp.int32(0)],
                                  csem.at[jnp.int32(0)]).wait()

        @pl.when(w1)
        def _():
            pltpu.make_async_copy(h2_hbm.at[pl.ds(0, _STG)],
                                  stg_v.at[jnp.int32(1)],
                                  csem.at[jnp.int32(1)]).wait()

    return gather_kernel


_ZERO = np.int32(0)


def _mm_body(h_ref, w_ref, scale_ref, o_ref):
    o_ref[...] = lax.dot_general(
        h_ref[...][:, :w_ref.shape[1]], w_ref[...],
        dimension_numbers=(((1,), (1,)), ((), ())),
        preferred_element_type=jnp.float32) * scale_ref[0]


def _make_matmul(n_tok, mdim, model_dim, block_rows=1024):
    grid = n_tok // block_rows
    return pl.pallas_call(
        _mm_body,
        grid=(grid,),
        in_specs=[
            pl.BlockSpec((block_rows, mdim), lambda i: (i, _ZERO)),
            pl.BlockSpec((model_dim, mdim // 2), lambda i: (_ZERO, _ZERO)),
            pl.BlockSpec((1,), lambda i: (_ZERO,), memory_space=pltpu.SMEM),
        ],
        out_specs=pl.BlockSpec((block_rows, model_dim), lambda i: (i, _ZERO)),
        out_shape=jax.ShapeDtypeStruct((n_tok, model_dim), jnp.float32),
    )


def kernel(token_ids, table, W_proj, scale):
    batch, seq = token_ids.shape
    vocab, dim = table.shape
    model_dim = W_proj.shape[0]
    n_tok = batch * seq

    tok = token_ids.astype(jnp.int32).reshape(-1)
    tok_prev = jnp.concatenate([jnp.zeros((1,), jnp.int32), tok[:-1]])
    tableT = table.T
    n_slabs = (vocab + _SLAB - 1) // _SLAB
    tlast = tableT[:, (n_slabs - 1) * _SLAB:]

    h2 = _make_gather(n_tok, vocab, dim, seq)(tok, tok_prev, tableT, tlast)
    scale1 = jnp.reshape(scale, (1,)).astype(jnp.float32)
    out = _make_matmul(n_tok, 2 * dim, model_dim)(h2[:n_tok], W_proj, scale1)
    return out.reshape(batch, seq, model_dim)
